# baseline (speedup 1.0000x reference)
import functools

import jax
import jax.numpy as jnp
from jax import lax
from jax.experimental import pallas as pl
from jax.experimental.pallas import tpu as pltpu
from jax.experimental.pallas import tpu_sc as plsc

N, E, IN, H, OUT = 10000, 320000, 128, 256, 64

NSUB = 16
NCORE = 2
FW = 128
K = 100
NBUF = 3
NBLK = NCORE * NSUB
CPB = E // NBLK // K
SUB = 20
NSTG = CPB // SUB
NP = 10240
RPT = NP // NSUB
ZR = 16
DEGW = 16


def _zero_rows(ref, rows, width):
    def zrow(i, c):
        for k in range(width // 16):
            ref[i, pl.ds(16 * k, 16)] = jnp.zeros((16,), jnp.float32)
        return c

    lax.fori_loop(0, rows, zrow, 0)


def _segsum_common(sid, blocks, feat, srcs, dsts, src_v, dst_v,
                   rows, acc, sems):
    _zero_rows(rows[0], K, FW)
    for i in range(RPT // K):
        pltpu.sync_copy(rows[0], acc.at[pl.ds(sid * RPT + i * K, K)])
    rem = RPT - (RPT // K) * K
    if rem:
        pltpu.sync_copy(rows[0].at[pl.ds(0, rem)],
                        acc.at[pl.ds(sid * RPT + (RPT // K) * K, rem)])

    plsc.subcore_barrier()

    gs, ss = sems[:NBUF], sems[NBUF:]

    def g_start(j):
        return pltpu.async_copy(feat.at[src_v.at[j]], rows[j % NBUF],
                                gs[j % NBUF])

    for b in blocks:
        def stage(st, c):
            pltpu.sync_copy(srcs.at[b * NSTG + st], src_v)
            pltpu.sync_copy(dsts.at[b * NSTG + st], dst_v)
            g = [None] * SUB
            s = [None] * SUB
            g[0] = g_start(0)
            g[1] = g_start(1)
            for j in range(SUB):
                B = j % NBUF
                g[j].wait()
                s[j] = pltpu.async_copy(rows[B], acc.at[dst_v.at[j]],
                                        ss[B], add=True)
                if j + 2 < SUB:
                    if j >= 1:
                        s[j - 1].wait()
                    g[j + 2] = g_start(j + 2)
            for j in range(max(0, SUB - NBUF), SUB):
                s[j].wait()
            return c

        lax.fori_loop(0, NSTG, stage, 0)
    plsc.subcore_barrier()


@functools.lru_cache(maxsize=None)
def _make_edge_split():
    outs = [jax.ShapeDtypeStruct((NP, FW), jnp.float32)] * 2
    scratch = [
        pltpu.VMEM((SUB, K), jnp.int32),
        pltpu.VMEM((SUB, K), jnp.int32),
    ] + [pltpu.VMEM((K, FW), jnp.float32)] * NBUF + [
        pltpu.VMEM_SHARED((NP, FW), jnp.float32),
    ] + [pltpu.SemaphoreType.DMA] * (2 * NBUF)
    mesh = plsc.VectorSubcoreMesh(core_axis_name="c", subcore_axis_name="s")

    def body(feat, srcs, dsts, pA, pB, src_v, dst_v, *rest):
        rows, acc, sems = rest[:NBUF], rest[NBUF], rest[NBUF + 1:]
        cid = lax.axis_index("c")
        sid = lax.axis_index("s")
        tid = cid * NSUB + sid
        _segsum_common(sid, [tid], feat, srcs, dsts, src_v, dst_v,
                       rows, acc, sems)
        sl = pl.ds(sid * RPT, RPT)

        @pl.when(cid == 0)
        def _():
            pltpu.sync_copy(acc.at[sl], pA.at[sl])

        @pl.when(cid == 1)
        def _():
            pltpu.sync_copy(acc.at[sl], pB.at[sl])

    return pl.kernel(body, out_type=outs, mesh=mesh, scratch_types=scratch,
                     compiler_params=pltpu.CompilerParams(use_tc_tiling_on_sc=False))


@functools.lru_cache(maxsize=None)
def _make_deg():
    outs = [jax.ShapeDtypeStruct((NP, DEGW), jnp.float32)] * 2
    scratch = [
        pltpu.VMEM((SUB, K), jnp.int32),
        pltpu.VMEM((ZR, DEGW), jnp.float32),
        pltpu.VMEM((K, DEGW), jnp.float32),
        pltpu.VMEM_SHARED((NP, DEGW), jnp.float32),
    ]
    mesh = plsc.VectorSubcoreMesh(core_axis_name="c", subcore_axis_name="s")

    def body(dsts, dA, dB, dst_v, zd, ones, dacc):
        cid = lax.axis_index("c")
        sid = lax.axis_index("s")
        tid = cid * NSUB + sid
        _zero_rows(zd, ZR, DEGW)

        def onerow(i, c):
            ones[i, pl.ds(0, DEGW)] = jnp.ones((DEGW,), jnp.float32)
            return c

        lax.fori_loop(0, K, onerow, 0)
        for i in range(RPT // ZR):
            pltpu.sync_copy(zd, dacc.at[pl.ds(sid * RPT + i * ZR, ZR)])
        plsc.subcore_barrier()

        def stage(st, c):
            pltpu.sync_copy(dsts.at[tid * NSTG + st], dst_v)

            def step(j, cc):
                pltpu.sync_copy(ones, dacc.at[dst_v.at[j]], add=True)
                return cc

            lax.fori_loop(0, SUB, step, 0)
            return c

        lax.fori_loop(0, NSTG, stage, 0)
        plsc.subcore_barrier()
        sl = pl.ds(sid * RPT, RPT)

        @pl.when(cid == 0)
        def _():
            pltpu.sync_copy(dacc.at[sl], dA.at[sl])

        @pl.when(cid == 1)
        def _():
            pltpu.sync_copy(dacc.at[sl], dB.at[sl])

    return pl.kernel(body, out_type=outs, mesh=mesh, scratch_types=scratch,
                     compiler_params=pltpu.CompilerParams(use_tc_tiling_on_sc=False))


@functools.lru_cache(maxsize=None)
def _make_feat_split():
    outs = [jax.ShapeDtypeStruct((NP, FW), jnp.float32)] * 2
    scratch = [
        pltpu.VMEM((SUB, K), jnp.int32),
        pltpu.VMEM((SUB, K), jnp.int32),
    ] + [pltpu.VMEM((K, FW), jnp.float32)] * NBUF + [
        pltpu.VMEM_SHARED((NP, FW), jnp.float32),
    ] + [pltpu.SemaphoreType.DMA] * (2 * NBUF)
    mesh = plsc.VectorSubcoreMesh(core_axis_name="c", subcore_axis_name="s")

    def body(featA, featB, srcs, dsts, aggA, aggB,
             src_v, dst_v, *rest):
        rows, acc, sems = rest[:NBUF], rest[NBUF], rest[NBUF + 1:]
        cid = lax.axis_index("c")
        sid = lax.axis_index("s")
        blocks = [2 * sid, 2 * sid + 1]

        @pl.when(cid == 0)
        def _():
            _segsum_common(sid, blocks, featA, srcs, dsts, src_v,
                           dst_v, rows, acc, sems)
            sl = pl.ds(sid * RPT, RPT)
            pltpu.sync_copy(acc.at[sl], aggA.at[sl])

        @pl.when(cid == 1)
        def _():
            _segsum_common(sid, blocks, featB, srcs, dsts, src_v,
                           dst_v, rows, acc, sems)
            sl = pl.ds(sid * RPT, RPT)
            pltpu.sync_copy(acc.at[sl], aggB.at[sl])

    return pl.kernel(body, out_type=outs, mesh=mesh, scratch_types=scratch,
                     compiler_params=pltpu.CompilerParams(use_tc_tiling_on_sc=False))


def _segsum1(*a):
    return _make_edge_split()(*a)


def _segsum2(*a):
    return _make_feat_split()(*a)


def _segsum3(*a):
    return _make_edge_split()(*a)


def _degcount(*a):
    return _make_deg()(*a)


BR = 400
GRID = N // BR


def _rep(shape):
    return pl.BlockSpec(shape, lambda i: (0,) * len(shape))


def _blk(c):
    return pl.BlockSpec((BR, c), lambda i: (i, 0))


def _rdeg(dA, dB):
    return 1.0 / jnp.maximum(dA[:, :1] + dB[:, :1], 1.0)


def _dotT(a, w):
    return lax.dot_general(a, w, (((1,), (1,)), ((), ())),
                           preferred_element_type=jnp.float32)


def _l1_body(pA, pB, dA, dB, xx, Wl, Wr, b, g, be, m, v, hA, hB):
    agg = pA[...] + pB[...]
    h = (_dotT(agg * _rdeg(dA, dB), Wl[...])
         + b[...]
         + _dotT(xx[...], Wr[...]))
    nrm = jnp.sqrt(jnp.sum(h * h, axis=1, keepdims=True))
    h = h / jnp.maximum(nrm, 1e-12)
    s = g[...] * lax.rsqrt(v[...] + 1e-5)
    h = jnp.maximum((h - m[...]) * s + be[...], 0.0)
    hA[...] = h[:, :FW]
    hB[...] = h[:, FW:]


def _l2_body(aA, aB, dA, dB, hA, hB, W2l, W2r, b2, g2, be2, m2, v2,
             W3l, W3r, b3, zP, r):
    agg = jnp.concatenate([aA[...], aB[...]], axis=1)
    xx = jnp.concatenate([hA[...], hB[...]], axis=1)
    h = (_dotT(agg * _rdeg(dA, dB), W2l[...])
         + b2[...]
         + _dotT(xx, W2r[...]))
    nrm = jnp.sqrt(jnp.sum(h * h, axis=1, keepdims=True))
    h = h / jnp.maximum(nrm, 1e-12)
    s = g2[...] * lax.rsqrt(v2[...] + 1e-5)
    h = jnp.maximum((h - m2[...]) * s + be2[...], 0.0)
    z = _dotT(h, W3l[...])
    zP[...] = jnp.concatenate(
        [z, jnp.zeros((z.shape[0], FW - OUT), jnp.float32)], axis=1)
    r[...] = _dotT(h, W3r[...]) + b3[...]


def _final_body(pA, pB, dA, dB, r, o):
    a3 = (pA[...] + pB[...])[:, :OUT]
    h = a3 * _rdeg(dA, dB) + r[...]
    nrm = jnp.sqrt(jnp.sum(h * h, axis=1, keepdims=True))
    h = h / jnp.maximum(nrm, 1e-12)
    mx = jnp.max(h, axis=1, keepdims=True)
    ex = jnp.exp(h - mx)
    o[...] = (h - mx) - jnp.log(jnp.sum(ex, axis=1, keepdims=True))


def kernel(x, edge_index, W1l, b1, W1r, g1, be1, m1, v1,
           W2l, b2, W2r, g2, be2, m2, v2, W3l, b3, W3r):
    srcE = edge_index[0].astype(jnp.int32).reshape(NBLK * NSTG, SUB, K)
    dstE = edge_index[1].astype(jnp.int32).reshape(NBLK * NSTG, SUB, K)

    r2 = lambda a: a.reshape(1, -1)

    dA, dB = _degcount(dstE)
    p1A, p1B = _segsum1(x, srcE, dstE)
    h1A, h1B = pl.pallas_call(
        _l1_body,
        grid=(GRID,),
        in_specs=[_blk(FW), _blk(FW), _blk(DEGW), _blk(DEGW), _blk(IN),
                  _rep((H, IN)), _rep((H, IN)),
                  _rep((1, H)), _rep((1, H)), _rep((1, H)), _rep((1, H)),
                  _rep((1, H))],
        out_specs=[_blk(FW), _blk(FW)],
        out_shape=[jax.ShapeDtypeStruct((N, FW), jnp.float32),
                   jax.ShapeDtypeStruct((N, FW), jnp.float32)],
    )(p1A, p1B, dA, dB, x, W1l, W1r, r2(b1), r2(g1), r2(be1), r2(m1),
      r2(v1))

    a2A, a2B = _segsum2(h1A, h1B, srcE, dstE)
    zP, r = pl.pallas_call(
        _l2_body,
        grid=(GRID,),
        in_specs=[_blk(FW), _blk(FW), _blk(DEGW), _blk(DEGW),
                  _blk(FW), _blk(FW),
                  _rep((H, H)), _rep((H, H)),
                  _rep((1, H)), _rep((1, H)), _rep((1, H)), _rep((1, H)),
                  _rep((1, H)),
                  _rep((OUT, H)), _rep((OUT, H)), _rep((1, OUT))],
        out_specs=[_blk(FW), _blk(OUT)],
        out_shape=[jax.ShapeDtypeStruct((N, FW), jnp.float32),
                   jax.ShapeDtypeStruct((N, OUT), jnp.float32)],
    )(a2A, a2B, dA, dB, h1A, h1B, W2l, W2r, r2(b2), r2(g2), r2(be2),
      r2(m2), r2(v2), W3l, W3r, r2(b3))

    p3A, p3B = _segsum3(zP, srcE, dstE)
    out = pl.pallas_call(
        _final_body,
        grid=(GRID,),
        in_specs=[_blk(FW), _blk(FW), _blk(DEGW), _blk(DEGW), _blk(OUT)],
        out_specs=_blk(OUT),
        out_shape=jax.ShapeDtypeStruct((N, OUT), jnp.float32),
    )(p3A, p3B, dA, dB, r)
    return out


# speedup vs baseline: 1.0142x; 1.0142x over previous
import functools

import jax
import jax.numpy as jnp
from jax import lax
from jax.experimental import pallas as pl
from jax.experimental.pallas import tpu as pltpu
from jax.experimental.pallas import tpu_sc as plsc

N, E, IN, H, OUT = 10000, 320000, 128, 256, 64

NSUB = 16
NCORE = 2
FW = 128
K = 100
NBUF = 3
NBLK = NCORE * NSUB
CPB = E // NBLK // K
SUB = 20
NSTG = CPB // SUB
NP = 10240
RPT = NP // NSUB
ZR = 16
DEGW = 16


def _zero_rows(ref, rows, width):
    def zrow(i, c):
        for k in range(width // 16):
            ref[i, pl.ds(16 * k, 16)] = jnp.zeros((16,), jnp.float32)
        return c

    lax.fori_loop(0, rows, zrow, 0)


def _segsum_common(sid, blocks, feat, srcs, dsts, src_v, dst_v,
                   rows, acc, sems):
    _zero_rows(rows[0], K, FW)
    for i in range(RPT // K):
        pltpu.sync_copy(rows[0], acc.at[pl.ds(sid * RPT + i * K, K)])
    rem = RPT - (RPT // K) * K
    if rem:
        pltpu.sync_copy(rows[0].at[pl.ds(0, rem)],
                        acc.at[pl.ds(sid * RPT + (RPT // K) * K, rem)])

    plsc.subcore_barrier()

    def g_start(j):
        return pltpu.async_copy(feat.at[src_v.at[j]], rows[j % NBUF],
                                sems[j % NBUF])

    for b in blocks:
        def stage(st, c):
            pltpu.sync_copy(srcs.at[b * NSTG + st], src_v)
            pltpu.sync_copy(dsts.at[b * NSTG + st], dst_v)
            g = [None] * SUB
            g[0] = g_start(0)
            g[1] = g_start(1)
            for j in range(SUB):
                if j + 2 < SUB:
                    g[j + 2] = g_start(j + 2)
                g[j].wait()
                pltpu.sync_copy(rows[j % NBUF], acc.at[dst_v.at[j]], add=True)
            return c

        lax.fori_loop(0, NSTG, stage, 0)
    plsc.subcore_barrier()


@functools.lru_cache(maxsize=None)
def _make_edge_split():
    outs = [jax.ShapeDtypeStruct((NP, FW), jnp.float32)] * 2
    scratch = [
        pltpu.VMEM((SUB, K), jnp.int32),
        pltpu.VMEM((SUB, K), jnp.int32),
    ] + [pltpu.VMEM((K, FW), jnp.float32)] * NBUF + [
        pltpu.VMEM_SHARED((NP, FW), jnp.float32),
    ] + [pltpu.SemaphoreType.DMA] * (2 * NBUF)
    mesh = plsc.VectorSubcoreMesh(core_axis_name="c", subcore_axis_name="s")

    def body(feat, srcs, dsts, pA, pB, src_v, dst_v, *rest):
        rows, acc, sems = rest[:NBUF], rest[NBUF], rest[NBUF + 1:]
        cid = lax.axis_index("c")
        sid = lax.axis_index("s")
        tid = cid * NSUB + sid
        _segsum_common(sid, [tid], feat, srcs, dsts, src_v, dst_v,
                       rows, acc, sems)
        sl = pl.ds(sid * RPT, RPT)

        @pl.when(cid == 0)
        def _():
            pltpu.sync_copy(acc.at[sl], pA.at[sl])

        @pl.when(cid == 1)
        def _():
            pltpu.sync_copy(acc.at[sl], pB.at[sl])

    return pl.kernel(body, out_type=outs, mesh=mesh, scratch_types=scratch,
                     compiler_params=pltpu.CompilerParams(use_tc_tiling_on_sc=False))


@functools.lru_cache(maxsize=None)
def _make_deg():
    outs = [jax.ShapeDtypeStruct((NP, DEGW), jnp.float32)] * 2
    scratch = [
        pltpu.VMEM((SUB, K), jnp.int32),
        pltpu.VMEM((ZR, DEGW), jnp.float32),
        pltpu.VMEM((K, DEGW), jnp.float32),
        pltpu.VMEM_SHARED((NP, DEGW), jnp.float32),
    ]
    mesh = plsc.VectorSubcoreMesh(core_axis_name="c", subcore_axis_name="s")

    def body(dsts, dA, dB, dst_v, zd, ones, dacc):
        cid = lax.axis_index("c")
        sid = lax.axis_index("s")
        tid = cid * NSUB + sid
        _zero_rows(zd, ZR, DEGW)

        def onerow(i, c):
            ones[i, pl.ds(0, DEGW)] = jnp.ones((DEGW,), jnp.float32)
            return c

        lax.fori_loop(0, K, onerow, 0)
        for i in range(RPT // ZR):
            pltpu.sync_copy(zd, dacc.at[pl.ds(sid * RPT + i * ZR, ZR)])
        plsc.subcore_barrier()

        def stage(st, c):
            pltpu.sync_copy(dsts.at[tid * NSTG + st], dst_v)

            def step(j, cc):
                pltpu.sync_copy(ones, dacc.at[dst_v.at[j]], add=True)
                return cc

            lax.fori_loop(0, SUB, step, 0)
            return c

        lax.fori_loop(0, NSTG, stage, 0)
        plsc.subcore_barrier()
        sl = pl.ds(sid * RPT, RPT)

        @pl.when(cid == 0)
        def _():
            pltpu.sync_copy(dacc.at[sl], dA.at[sl])

        @pl.when(cid == 1)
        def _():
            pltpu.sync_copy(dacc.at[sl], dB.at[sl])

    return pl.kernel(body, out_type=outs, mesh=mesh, scratch_types=scratch,
                     compiler_params=pltpu.CompilerParams(use_tc_tiling_on_sc=False))


@functools.lru_cache(maxsize=None)
def _make_feat_split():
    outs = [jax.ShapeDtypeStruct((NP, FW), jnp.float32)] * 2
    scratch = [
        pltpu.VMEM((SUB, K), jnp.int32),
        pltpu.VMEM((SUB, K), jnp.int32),
    ] + [pltpu.VMEM((K, FW), jnp.float32)] * NBUF + [
        pltpu.VMEM_SHARED((NP, FW), jnp.float32),
    ] + [pltpu.SemaphoreType.DMA] * (2 * NBUF)
    mesh = plsc.VectorSubcoreMesh(core_axis_name="c", subcore_axis_name="s")

    def body(featA, featB, srcs, dsts, aggA, aggB,
             src_v, dst_v, *rest):
        rows, acc, sems = rest[:NBUF], rest[NBUF], rest[NBUF + 1:]
        cid = lax.axis_index("c")
        sid = lax.axis_index("s")
        blocks = [2 * sid, 2 * sid + 1]

        @pl.when(cid == 0)
        def _():
            _segsum_common(sid, blocks, featA, srcs, dsts, src_v,
                           dst_v, rows, acc, sems)
            sl = pl.ds(sid * RPT, RPT)
            pltpu.sync_copy(acc.at[sl], aggA.at[sl])

        @pl.when(cid == 1)
        def _():
            _segsum_common(sid, blocks, featB, srcs, dsts, src_v,
                           dst_v, rows, acc, sems)
            sl = pl.ds(sid * RPT, RPT)
            pltpu.sync_copy(acc.at[sl], aggB.at[sl])

    return pl.kernel(body, out_type=outs, mesh=mesh, scratch_types=scratch,
                     compiler_params=pltpu.CompilerParams(use_tc_tiling_on_sc=False))


def _segsum1(*a):
    return _make_edge_split()(*a)


def _segsum2(*a):
    return _make_feat_split()(*a)


def _segsum3(*a):
    return _make_edge_split()(*a)


def _degcount(*a):
    return _make_deg()(*a)


BR = 400
GRID = N // BR


def _rep(shape):
    return pl.BlockSpec(shape, lambda i: (0,) * len(shape))


def _blk(c):
    return pl.BlockSpec((BR, c), lambda i: (i, 0))


def _rdeg(dA, dB):
    return 1.0 / jnp.maximum(dA[:, :1] + dB[:, :1], 1.0)


def _dotT(a, w):
    return lax.dot_general(a, w, (((1,), (1,)), ((), ())),
                           preferred_element_type=jnp.float32)


def _l1_body(pA, pB, dA, dB, xx, Wl, Wr, b, g, be, m, v, hA, hB):
    agg = pA[...] + pB[...]
    h = (_dotT(agg * _rdeg(dA, dB), Wl[...])
         + b[...]
         + _dotT(xx[...], Wr[...]))
    nrm = jnp.sqrt(jnp.sum(h * h, axis=1, keepdims=True))
    h = h / jnp.maximum(nrm, 1e-12)
    s = g[...] * lax.rsqrt(v[...] + 1e-5)
    h = jnp.maximum((h - m[...]) * s + be[...], 0.0)
    hA[...] = h[:, :FW]
    hB[...] = h[:, FW:]


def _l2_body(aA, aB, dA, dB, hA, hB, W2l, W2r, b2, g2, be2, m2, v2,
             W3l, W3r, b3, zP, r):
    agg = jnp.concatenate([aA[...], aB[...]], axis=1)
    xx = jnp.concatenate([hA[...], hB[...]], axis=1)
    h = (_dotT(agg * _rdeg(dA, dB), W2l[...])
         + b2[...]
         + _dotT(xx, W2r[...]))
    nrm = jnp.sqrt(jnp.sum(h * h, axis=1, keepdims=True))
    h = h / jnp.maximum(nrm, 1e-12)
    s = g2[...] * lax.rsqrt(v2[...] + 1e-5)
    h = jnp.maximum((h - m2[...]) * s + be2[...], 0.0)
    z = _dotT(h, W3l[...])
    zP[...] = jnp.concatenate(
        [z, jnp.zeros((z.shape[0], FW - OUT), jnp.float32)], axis=1)
    r[...] = _dotT(h, W3r[...]) + b3[...]


def _final_body(pA, pB, dA, dB, r, o):
    a3 = (pA[...] + pB[...])[:, :OUT]
    h = a3 * _rdeg(dA, dB) + r[...]
    nrm = jnp.sqrt(jnp.sum(h * h, axis=1, keepdims=True))
    h = h / jnp.maximum(nrm, 1e-12)
    mx = jnp.max(h, axis=1, keepdims=True)
    ex = jnp.exp(h - mx)
    o[...] = (h - mx) - jnp.log(jnp.sum(ex, axis=1, keepdims=True))


def kernel(x, edge_index, W1l, b1, W1r, g1, be1, m1, v1,
           W2l, b2, W2r, g2, be2, m2, v2, W3l, b3, W3r):
    srcE = edge_index[0].astype(jnp.int32).reshape(NBLK * NSTG, SUB, K)
    dstE = edge_index[1].astype(jnp.int32).reshape(NBLK * NSTG, SUB, K)

    r2 = lambda a: a.reshape(1, -1)

    dA, dB = _degcount(dstE)
    p1A, p1B = _segsum1(x, srcE, dstE)
    h1A, h1B = pl.pallas_call(
        _l1_body,
        grid=(GRID,),
        in_specs=[_blk(FW), _blk(FW), _blk(DEGW), _blk(DEGW), _blk(IN),
                  _rep((H, IN)), _rep((H, IN)),
                  _rep((1, H)), _rep((1, H)), _rep((1, H)), _rep((1, H)),
                  _rep((1, H))],
        out_specs=[_blk(FW), _blk(FW)],
        out_shape=[jax.ShapeDtypeStruct((N, FW), jnp.float32),
                   jax.ShapeDtypeStruct((N, FW), jnp.float32)],
    )(p1A, p1B, dA, dB, x, W1l, W1r, r2(b1), r2(g1), r2(be1), r2(m1),
      r2(v1))

    a2A, a2B = _segsum2(h1A, h1B, srcE, dstE)
    zP, r = pl.pallas_call(
        _l2_body,
        grid=(GRID,),
        in_specs=[_blk(FW), _blk(FW), _blk(DEGW), _blk(DEGW),
                  _blk(FW), _blk(FW),
                  _rep((H, H)), _rep((H, H)),
                  _rep((1, H)), _rep((1, H)), _rep((1, H)), _rep((1, H)),
                  _rep((1, H)),
                  _rep((OUT, H)), _rep((OUT, H)), _rep((1, OUT))],
        out_specs=[_blk(FW), _blk(OUT)],
        out_shape=[jax.ShapeDtypeStruct((N, FW), jnp.float32),
                   jax.ShapeDtypeStruct((N, OUT), jnp.float32)],
    )(a2A, a2B, dA, dB, h1A, h1B, W2l, W2r, r2(b2), r2(g2), r2(be2),
      r2(m2), r2(v2), W3l, W3r, r2(b3))

    p3A, p3B = _segsum3(zP, srcE, dstE)
    out = pl.pallas_call(
        _final_body,
        grid=(GRID,),
        in_specs=[_blk(FW), _blk(FW), _blk(DEGW), _blk(DEGW), _blk(OUT)],
        out_specs=_blk(OUT),
        out_shape=jax.ShapeDtypeStruct((N, OUT), jnp.float32),
    )(p3A, p3B, dA, dB, r)
    return out


# speedup vs baseline: 1.0364x; 1.0219x over previous
import functools

import jax
import jax.numpy as jnp
from jax import lax
from jax.experimental import pallas as pl
from jax.experimental.pallas import tpu as pltpu
from jax.experimental.pallas import tpu_sc as plsc

N, E, IN, H, OUT = 10000, 320000, 128, 256, 64

NSUB = 16
NCORE = 2
FW = 128
K = 100
NBUF = 3
NBLK = NCORE * NSUB
CPB = E // NBLK // K
SUB = 25
NSTG = CPB // SUB
NP = 10240
RPT = NP // NSUB
ZR = 16
DEGW = 16


def _zero_rows(ref, rows, width):
    def zrow(i, c):
        for k in range(width // 16):
            ref[i, pl.ds(16 * k, 16)] = jnp.zeros((16,), jnp.float32)
        return c

    lax.fori_loop(0, rows, zrow, 0)


def _segsum_common(sid, blocks, feat, srcs, dsts, src_v, dst_v,
                   rows, acc, sems):
    _zero_rows(rows[0], K, FW)
    rem = RPT - (RPT // K) * K
    blit_dsts = [acc.at[pl.ds(sid * RPT + i * K, K)]
                 for i in range(RPT // K)]
    last = [None] * NBUF
    for i, dstref in enumerate(blit_dsts):
        sidx = i % NBUF
        if last[sidx] is not None:
            last[sidx].wait()
        last[sidx] = pltpu.async_copy(rows[0], dstref, sems[sidx])
    if rem:
        sidx = len(blit_dsts) % NBUF
        if last[sidx] is not None:
            last[sidx].wait()
        last[sidx] = pltpu.async_copy(
            rows[0].at[pl.ds(0, rem)],
            acc.at[pl.ds(sid * RPT + (RPT // K) * K, rem)], sems[sidx])
    for d in last:
        if d is not None:
            d.wait()

    plsc.subcore_barrier()

    def g_start(j):
        return pltpu.async_copy(feat.at[src_v.at[j]], rows[j % NBUF],
                                sems[j % NBUF])

    for b in blocks:
        def stage(st, c):
            pltpu.sync_copy(srcs.at[b * NSTG + st], src_v)
            pltpu.sync_copy(dsts.at[b * NSTG + st], dst_v)
            g = [None] * SUB
            g[0] = g_start(0)
            g[1] = g_start(1)
            for j in range(SUB):
                if j + 2 < SUB:
                    g[j + 2] = g_start(j + 2)
                g[j].wait()
                pltpu.sync_copy(rows[j % NBUF], acc.at[dst_v.at[j]], add=True)
            return c

        lax.fori_loop(0, NSTG, stage, 0)
    plsc.subcore_barrier()


@functools.lru_cache(maxsize=None)
def _make_edge_split():
    outs = [jax.ShapeDtypeStruct((NP, FW), jnp.float32)] * 2
    scratch = [
        pltpu.VMEM((SUB, K), jnp.int32),
        pltpu.VMEM((SUB, K), jnp.int32),
    ] + [pltpu.VMEM((K, FW), jnp.float32)] * NBUF + [
        pltpu.VMEM_SHARED((NP, FW), jnp.float32),
    ] + [pltpu.SemaphoreType.DMA] * (2 * NBUF)
    mesh = plsc.VectorSubcoreMesh(core_axis_name="c", subcore_axis_name="s")

    def body(feat, srcs, dsts, pA, pB, src_v, dst_v, *rest):
        rows, acc, sems = rest[:NBUF], rest[NBUF], rest[NBUF + 1:]
        cid = lax.axis_index("c")
        sid = lax.axis_index("s")
        tid = cid * NSUB + sid
        _segsum_common(sid, [tid], feat, srcs, dsts, src_v, dst_v,
                       rows, acc, sems)
        sl = pl.ds(sid * RPT, RPT)

        @pl.when(cid == 0)
        def _():
            pltpu.sync_copy(acc.at[sl], pA.at[sl])

        @pl.when(cid == 1)
        def _():
            pltpu.sync_copy(acc.at[sl], pB.at[sl])

    return pl.kernel(body, out_type=outs, mesh=mesh, scratch_types=scratch,
                     compiler_params=pltpu.CompilerParams(use_tc_tiling_on_sc=False))


@functools.lru_cache(maxsize=None)
def _make_deg():
    outs = [jax.ShapeDtypeStruct((NP, DEGW), jnp.float32)] * 2
    scratch = [
        pltpu.VMEM((SUB, K), jnp.int32),
        pltpu.VMEM((ZR, DEGW), jnp.float32),
        pltpu.VMEM((K, DEGW), jnp.float32),
        pltpu.VMEM_SHARED((NP, DEGW), jnp.float32),
    ]
    mesh = plsc.VectorSubcoreMesh(core_axis_name="c", subcore_axis_name="s")

    def body(dsts, dA, dB, dst_v, zd, ones, dacc):
        cid = lax.axis_index("c")
        sid = lax.axis_index("s")
        tid = cid * NSUB + sid
        _zero_rows(zd, ZR, DEGW)

        def onerow(i, c):
            ones[i, pl.ds(0, DEGW)] = jnp.ones((DEGW,), jnp.float32)
            return c

        lax.fori_loop(0, K, onerow, 0)
        for i in range(RPT // ZR):
            pltpu.sync_copy(zd, dacc.at[pl.ds(sid * RPT + i * ZR, ZR)])
        plsc.subcore_barrier()

        def stage(st, c):
            pltpu.sync_copy(dsts.at[tid * NSTG + st], dst_v)

            def step(j, cc):
                pltpu.sync_copy(ones, dacc.at[dst_v.at[j]], add=True)
                return cc

            lax.fori_loop(0, SUB, step, 0)
            return c

        lax.fori_loop(0, NSTG, stage, 0)
        plsc.subcore_barrier()
        sl = pl.ds(sid * RPT, RPT)

        @pl.when(cid == 0)
        def _():
            pltpu.sync_copy(dacc.at[sl], dA.at[sl])

        @pl.when(cid == 1)
        def _():
            pltpu.sync_copy(dacc.at[sl], dB.at[sl])

    return pl.kernel(body, out_type=outs, mesh=mesh, scratch_types=scratch,
                     compiler_params=pltpu.CompilerParams(use_tc_tiling_on_sc=False))


@functools.lru_cache(maxsize=None)
def _make_feat_split():
    outs = [jax.ShapeDtypeStruct((NP, FW), jnp.float32)] * 2
    scratch = [
        pltpu.VMEM((SUB, K), jnp.int32),
        pltpu.VMEM((SUB, K), jnp.int32),
    ] + [pltpu.VMEM((K, FW), jnp.float32)] * NBUF + [
        pltpu.VMEM_SHARED((NP, FW), jnp.float32),
    ] + [pltpu.SemaphoreType.DMA] * (2 * NBUF)
    mesh = plsc.VectorSubcoreMesh(core_axis_name="c", subcore_axis_name="s")

    def body(featA, featB, srcs, dsts, aggA, aggB,
             src_v, dst_v, *rest):
        rows, acc, sems = rest[:NBUF], rest[NBUF], rest[NBUF + 1:]
        cid = lax.axis_index("c")
        sid = lax.axis_index("s")
        blocks = [2 * sid, 2 * sid + 1]

        @pl.when(cid == 0)
        def _():
            _segsum_common(sid, blocks, featA, srcs, dsts, src_v,
                           dst_v, rows, acc, sems)
            sl = pl.ds(sid * RPT, RPT)
            pltpu.sync_copy(acc.at[sl], aggA.at[sl])

        @pl.when(cid == 1)
        def _():
            _segsum_common(sid, blocks, featB, srcs, dsts, src_v,
                           dst_v, rows, acc, sems)
            sl = pl.ds(sid * RPT, RPT)
            pltpu.sync_copy(acc.at[sl], aggB.at[sl])

    return pl.kernel(body, out_type=outs, mesh=mesh, scratch_types=scratch,
                     compiler_params=pltpu.CompilerParams(use_tc_tiling_on_sc=False))


def _segsum1(*a):
    return _make_edge_split()(*a)


def _segsum2(*a):
    return _make_feat_split()(*a)


def _segsum3(*a):
    return _make_edge_split()(*a)


def _degcount(*a):
    return _make_deg()(*a)


BR = 400
GRID = N // BR


def _rep(shape):
    return pl.BlockSpec(shape, lambda i: (0,) * len(shape))


def _blk(c):
    return pl.BlockSpec((BR, c), lambda i: (i, 0))


def _rdeg(dA, dB):
    return 1.0 / jnp.maximum(dA[:, :1] + dB[:, :1], 1.0)


def _dotT(a, w):
    return lax.dot_general(a, w, (((1,), (1,)), ((), ())),
                           preferred_element_type=jnp.float32)


def _l1_body(pA, pB, dA, dB, xx, Wl, Wr, b, g, be, m, v, hA, hB):
    agg = pA[...] + pB[...]
    h = (_dotT(agg * _rdeg(dA, dB), Wl[...])
         + b[...]
         + _dotT(xx[...], Wr[...]))
    nrm = jnp.sqrt(jnp.sum(h * h, axis=1, keepdims=True))
    h = h / jnp.maximum(nrm, 1e-12)
    s = g[...] * lax.rsqrt(v[...] + 1e-5)
    h = jnp.maximum((h - m[...]) * s + be[...], 0.0)
    hA[...] = h[:, :FW]
    hB[...] = h[:, FW:]


def _l2_body(aA, aB, dA, dB, hA, hB, W2l, W2r, b2, g2, be2, m2, v2,
             W3l, W3r, b3, zP, r):
    agg = jnp.concatenate([aA[...], aB[...]], axis=1)
    xx = jnp.concatenate([hA[...], hB[...]], axis=1)
    h = (_dotT(agg * _rdeg(dA, dB), W2l[...])
         + b2[...]
         + _dotT(xx, W2r[...]))
    nrm = jnp.sqrt(jnp.sum(h * h, axis=1, keepdims=True))
    h = h / jnp.maximum(nrm, 1e-12)
    s = g2[...] * lax.rsqrt(v2[...] + 1e-5)
    h = jnp.maximum((h - m2[...]) * s + be2[...], 0.0)
    z = _dotT(h, W3l[...])
    zP[...] = jnp.concatenate(
        [z, jnp.zeros((z.shape[0], FW - OUT), jnp.float32)], axis=1)
    r[...] = _dotT(h, W3r[...]) + b3[...]


def _final_body(pA, pB, dA, dB, r, o):
    a3 = (pA[...] + pB[...])[:, :OUT]
    h = a3 * _rdeg(dA, dB) + r[...]
    nrm = jnp.sqrt(jnp.sum(h * h, axis=1, keepdims=True))
    h = h / jnp.maximum(nrm, 1e-12)
    mx = jnp.max(h, axis=1, keepdims=True)
    ex = jnp.exp(h - mx)
    o[...] = (h - mx) - jnp.log(jnp.sum(ex, axis=1, keepdims=True))


def kernel(x, edge_index, W1l, b1, W1r, g1, be1, m1, v1,
           W2l, b2, W2r, g2, be2, m2, v2, W3l, b3, W3r):
    srcE = edge_index[0].astype(jnp.int32).reshape(NBLK * NSTG, SUB, K)
    dstE = edge_index[1].astype(jnp.int32).reshape(NBLK * NSTG, SUB, K)

    r2 = lambda a: a.reshape(1, -1)

    dA, dB = _degcount(dstE)
    p1A, p1B = _segsum1(x, srcE, dstE)
    h1A, h1B = pl.pallas_call(
        _l1_body,
        grid=(GRID,),
        in_specs=[_blk(FW), _blk(FW), _blk(DEGW), _blk(DEGW), _blk(IN),
                  _rep((H, IN)), _rep((H, IN)),
                  _rep((1, H)), _rep((1, H)), _rep((1, H)), _rep((1, H)),
                  _rep((1, H))],
        out_specs=[_blk(FW), _blk(FW)],
        out_shape=[jax.ShapeDtypeStruct((N, FW), jnp.float32),
                   jax.ShapeDtypeStruct((N, FW), jnp.float32)],
    )(p1A, p1B, dA, dB, x, W1l, W1r, r2(b1), r2(g1), r2(be1), r2(m1),
      r2(v1))

    a2A, a2B = _segsum2(h1A, h1B, srcE, dstE)
    zP, r = pl.pallas_call(
        _l2_body,
        grid=(GRID,),
        in_specs=[_blk(FW), _blk(FW), _blk(DEGW), _blk(DEGW),
                  _blk(FW), _blk(FW),
                  _rep((H, H)), _rep((H, H)),
                  _rep((1, H)), _rep((1, H)), _rep((1, H)), _rep((1, H)),
                  _rep((1, H)),
                  _rep((OUT, H)), _rep((OUT, H)), _rep((1, OUT))],
        out_specs=[_blk(FW), _blk(OUT)],
        out_shape=[jax.ShapeDtypeStruct((N, FW), jnp.float32),
                   jax.ShapeDtypeStruct((N, OUT), jnp.float32)],
    )(a2A, a2B, dA, dB, h1A, h1B, W2l, W2r, r2(b2), r2(g2), r2(be2),
      r2(m2), r2(v2), W3l, W3r, r2(b3))

    p3A, p3B = _segsum3(zP, srcE, dstE)
    out = pl.pallas_call(
        _final_body,
        grid=(GRID,),
        in_specs=[_blk(FW), _blk(FW), _blk(DEGW), _blk(DEGW), _blk(OUT)],
        out_specs=_blk(OUT),
        out_shape=jax.ShapeDtypeStruct((N, OUT), jnp.float32),
    )(p3A, p3B, dA, dB, r)
    return out


# speedup vs baseline: 1.0417x; 1.0051x over previous
import functools

import jax
import jax.numpy as jnp
from jax import lax
from jax.experimental import pallas as pl
from jax.experimental.pallas import tpu as pltpu
from jax.experimental.pallas import tpu_sc as plsc

N, E, IN, H, OUT = 10000, 320000, 128, 256, 64

NSUB = 16
NCORE = 2
FW = 128
K = 100
NBUF = 3
NBLK = NCORE * NSUB
CPB = E // NBLK // K
SUB = 25
NSTG = CPB // SUB
NP = 10240
RPT = NP // NSUB
ZR = 16
DEGW = 16


def _zero_rows(ref, rows, width):
    def zrow(i, c):
        for k in range(width // 16):
            ref[i, pl.ds(16 * k, 16)] = jnp.zeros((16,), jnp.float32)
        return c

    lax.fori_loop(0, rows, zrow, 0)


def _segsum_common(sid, blocks, feat, srcs, dsts, src_v, dst_v,
                   rows, acc, sems):
    _zero_rows(rows[0], K, FW)
    rem = RPT - (RPT // K) * K
    blit_dsts = [acc.at[pl.ds(sid * RPT + i * K, K)]
                 for i in range(RPT // K)]
    last = [None] * NBUF
    for i, dstref in enumerate(blit_dsts):
        sidx = i % NBUF
        if last[sidx] is not None:
            last[sidx].wait()
        last[sidx] = pltpu.async_copy(rows[0], dstref, sems[sidx])
    if rem:
        sidx = len(blit_dsts) % NBUF
        if last[sidx] is not None:
            last[sidx].wait()
        last[sidx] = pltpu.async_copy(
            rows[0].at[pl.ds(0, rem)],
            acc.at[pl.ds(sid * RPT + (RPT // K) * K, rem)], sems[sidx])
    for d in last:
        if d is not None:
            d.wait()

    plsc.subcore_barrier()

    def g_start(j):
        return pltpu.async_copy(feat.at[src_v.at[j]], rows[j % NBUF],
                                sems[j % NBUF])

    base, nstages = blocks

    def stage(st, c):
        pltpu.sync_copy(srcs.at[base + st], src_v)
        pltpu.sync_copy(dsts.at[base + st], dst_v)
        g = [None] * SUB
        g[0] = g_start(0)
        g[1] = g_start(1)
        for j in range(SUB):
            if j + 2 < SUB:
                g[j + 2] = g_start(j + 2)
            g[j].wait()
            pltpu.sync_copy(rows[j % NBUF], acc.at[dst_v.at[j]], add=True)
        return c

    lax.fori_loop(0, nstages, stage, 0)
    plsc.subcore_barrier()


@functools.lru_cache(maxsize=None)
def _make_edge_split():
    outs = [jax.ShapeDtypeStruct((NP, FW), jnp.float32)] * 2
    scratch = [
        pltpu.VMEM((SUB, K), jnp.int32),
        pltpu.VMEM((SUB, K), jnp.int32),
    ] + [pltpu.VMEM((K, FW), jnp.float32)] * NBUF + [
        pltpu.VMEM_SHARED((NP, FW), jnp.float32),
    ] + [pltpu.SemaphoreType.DMA] * (2 * NBUF)
    mesh = plsc.VectorSubcoreMesh(core_axis_name="c", subcore_axis_name="s")

    def body(feat, srcs, dsts, pA, pB, src_v, dst_v, *rest):
        rows, acc, sems = rest[:NBUF], rest[NBUF], rest[NBUF + 1:]
        cid = lax.axis_index("c")
        sid = lax.axis_index("s")
        tid = cid * NSUB + sid
        _segsum_common(sid, (tid * NSTG, NSTG), feat, srcs, dsts, src_v,
                       dst_v, rows, acc, sems)
        sl = pl.ds(sid * RPT, RPT)

        @pl.when(cid == 0)
        def _():
            pltpu.sync_copy(acc.at[sl], pA.at[sl])

        @pl.when(cid == 1)
        def _():
            pltpu.sync_copy(acc.at[sl], pB.at[sl])

    return pl.kernel(body, out_type=outs, mesh=mesh, scratch_types=scratch,
                     compiler_params=pltpu.CompilerParams(use_tc_tiling_on_sc=False))


@functools.lru_cache(maxsize=None)
def _make_deg():
    outs = [jax.ShapeDtypeStruct((NP, DEGW), jnp.float32)] * 2
    scratch = [
        pltpu.VMEM((SUB, K), jnp.int32),
        pltpu.VMEM((ZR, DEGW), jnp.float32),
        pltpu.VMEM((K, DEGW), jnp.float32),
        pltpu.VMEM_SHARED((NP, DEGW), jnp.float32),
    ]
    mesh = plsc.VectorSubcoreMesh(core_axis_name="c", subcore_axis_name="s")

    def body(dsts, dA, dB, dst_v, zd, ones, dacc):
        cid = lax.axis_index("c")
        sid = lax.axis_index("s")
        tid = cid * NSUB + sid
        _zero_rows(zd, ZR, DEGW)

        def onerow(i, c):
            ones[i, pl.ds(0, DEGW)] = jnp.ones((DEGW,), jnp.float32)
            return c

        lax.fori_loop(0, K, onerow, 0)
        for i in range(RPT // ZR):
            pltpu.sync_copy(zd, dacc.at[pl.ds(sid * RPT + i * ZR, ZR)])
        plsc.subcore_barrier()

        def stage(st, c):
            pltpu.sync_copy(dsts.at[tid * NSTG + st], dst_v)

            def step(j, cc):
                pltpu.sync_copy(ones, dacc.at[dst_v.at[j]], add=True)
                return cc

            lax.fori_loop(0, SUB, step, 0)
            return c

        lax.fori_loop(0, NSTG, stage, 0)
        plsc.subcore_barrier()
        sl = pl.ds(sid * RPT, RPT)

        @pl.when(cid == 0)
        def _():
            pltpu.sync_copy(dacc.at[sl], dA.at[sl])

        @pl.when(cid == 1)
        def _():
            pltpu.sync_copy(dacc.at[sl], dB.at[sl])

    return pl.kernel(body, out_type=outs, mesh=mesh, scratch_types=scratch,
                     compiler_params=pltpu.CompilerParams(use_tc_tiling_on_sc=False))


@functools.lru_cache(maxsize=None)
def _make_feat_split():
    outs = [jax.ShapeDtypeStruct((NP, FW), jnp.float32)] * 2
    scratch = [
        pltpu.VMEM((SUB, K), jnp.int32),
        pltpu.VMEM((SUB, K), jnp.int32),
    ] + [pltpu.VMEM((K, FW), jnp.float32)] * NBUF + [
        pltpu.VMEM_SHARED((NP, FW), jnp.float32),
    ] + [pltpu.SemaphoreType.DMA] * (2 * NBUF)
    mesh = plsc.VectorSubcoreMesh(core_axis_name="c", subcore_axis_name="s")

    def body(featA, featB, srcs, dsts, aggA, aggB,
             src_v, dst_v, *rest):
        rows, acc, sems = rest[:NBUF], rest[NBUF], rest[NBUF + 1:]
        cid = lax.axis_index("c")
        sid = lax.axis_index("s")

        @pl.when(cid == 0)
        def _():
            _segsum_common(sid, (2 * sid * NSTG, 2 * NSTG), featA, srcs,
                           dsts, src_v, dst_v, rows, acc, sems)
            sl = pl.ds(sid * RPT, RPT)
            pltpu.sync_copy(acc.at[sl], aggA.at[sl])

        @pl.when(cid == 1)
        def _():
            _segsum_common(sid, (2 * sid * NSTG, 2 * NSTG), featB, srcs,
                           dsts, src_v, dst_v, rows, acc, sems)
            sl = pl.ds(sid * RPT, RPT)
            pltpu.sync_copy(acc.at[sl], aggB.at[sl])

    return pl.kernel(body, out_type=outs, mesh=mesh, scratch_types=scratch,
                     compiler_params=pltpu.CompilerParams(use_tc_tiling_on_sc=False))


def _segsum1(*a):
    return _make_edge_split()(*a)


def _segsum2(*a):
    return _make_feat_split()(*a)


def _segsum3(*a):
    return _make_edge_split()(*a)


def _degcount(*a):
    return _make_deg()(*a)


BR = 400
GRID = N // BR


def _rep(shape):
    return pl.BlockSpec(shape, lambda i: (0,) * len(shape))


def _blk(c):
    return pl.BlockSpec((BR, c), lambda i: (i, 0))


def _rdeg(dA, dB):
    return 1.0 / jnp.maximum(dA[:, :1] + dB[:, :1], 1.0)


def _dotT(a, w):
    return lax.dot_general(a, w, (((1,), (1,)), ((), ())),
                           preferred_element_type=jnp.float32)


def _l1_body(pA, pB, dA, dB, xx, Wl, Wr, b, g, be, m, v, hA, hB):
    agg = pA[...] + pB[...]
    h = (_dotT(agg * _rdeg(dA, dB), Wl[...])
         + b[...]
         + _dotT(xx[...], Wr[...]))
    nrm = jnp.sqrt(jnp.sum(h * h, axis=1, keepdims=True))
    h = h / jnp.maximum(nrm, 1e-12)
    s = g[...] * lax.rsqrt(v[...] + 1e-5)
    h = jnp.maximum((h - m[...]) * s + be[...], 0.0)
    hA[...] = h[:, :FW]
    hB[...] = h[:, FW:]


def _l2_body(aA, aB, dA, dB, hA, hB, W2l, W2r, b2, g2, be2, m2, v2,
             W3l, W3r, b3, zP, r):
    agg = jnp.concatenate([aA[...], aB[...]], axis=1)
    xx = jnp.concatenate([hA[...], hB[...]], axis=1)
    h = (_dotT(agg * _rdeg(dA, dB), W2l[...])
         + b2[...]
         + _dotT(xx, W2r[...]))
    nrm = jnp.sqrt(jnp.sum(h * h, axis=1, keepdims=True))
    h = h / jnp.maximum(nrm, 1e-12)
    s = g2[...] * lax.rsqrt(v2[...] + 1e-5)
    h = jnp.maximum((h - m2[...]) * s + be2[...], 0.0)
    z = _dotT(h, W3l[...])
    zP[...] = jnp.concatenate(
        [z, jnp.zeros((z.shape[0], FW - OUT), jnp.float32)], axis=1)
    r[...] = _dotT(h, W3r[...]) + b3[...]


def _final_body(pA, pB, dA, dB, r, o):
    a3 = (pA[...] + pB[...])[:, :OUT]
    h = a3 * _rdeg(dA, dB) + r[...]
    nrm = jnp.sqrt(jnp.sum(h * h, axis=1, keepdims=True))
    h = h / jnp.maximum(nrm, 1e-12)
    mx = jnp.max(h, axis=1, keepdims=True)
    ex = jnp.exp(h - mx)
    o[...] = (h - mx) - jnp.log(jnp.sum(ex, axis=1, keepdims=True))


def kernel(x, edge_index, W1l, b1, W1r, g1, be1, m1, v1,
           W2l, b2, W2r, g2, be2, m2, v2, W3l, b3, W3r):
    srcE = edge_index[0].astype(jnp.int32).reshape(NBLK * NSTG, SUB, K)
    dstE = edge_index[1].astype(jnp.int32).reshape(NBLK * NSTG, SUB, K)

    r2 = lambda a: a.reshape(1, -1)

    dA, dB = _degcount(dstE)
    p1A, p1B = _segsum1(x, srcE, dstE)
    h1A, h1B = pl.pallas_call(
        _l1_body,
        grid=(GRID,),
        in_specs=[_blk(FW), _blk(FW), _blk(DEGW), _blk(DEGW), _blk(IN),
                  _rep((H, IN)), _rep((H, IN)),
                  _rep((1, H)), _rep((1, H)), _rep((1, H)), _rep((1, H)),
                  _rep((1, H))],
        out_specs=[_blk(FW), _blk(FW)],
        out_shape=[jax.ShapeDtypeStruct((N, FW), jnp.float32),
                   jax.ShapeDtypeStruct((N, FW), jnp.float32)],
    )(p1A, p1B, dA, dB, x, W1l, W1r, r2(b1), r2(g1), r2(be1), r2(m1),
      r2(v1))

    a2A, a2B = _segsum2(h1A, h1B, srcE, dstE)
    zP, r = pl.pallas_call(
        _l2_body,
        grid=(GRID,),
        in_specs=[_blk(FW), _blk(FW), _blk(DEGW), _blk(DEGW),
                  _blk(FW), _blk(FW),
                  _rep((H, H)), _rep((H, H)),
                  _rep((1, H)), _rep((1, H)), _rep((1, H)), _rep((1, H)),
                  _rep((1, H)),
                  _rep((OUT, H)), _rep((OUT, H)), _rep((1, OUT))],
        out_specs=[_blk(FW), _blk(OUT)],
        out_shape=[jax.ShapeDtypeStruct((N, FW), jnp.float32),
                   jax.ShapeDtypeStruct((N, OUT), jnp.float32)],
    )(a2A, a2B, dA, dB, h1A, h1B, W2l, W2r, r2(b2), r2(g2), r2(be2),
      r2(m2), r2(v2), W3l, W3r, r2(b3))

    p3A, p3B = _segsum3(zP, srcE, dstE)
    out = pl.pallas_call(
        _final_body,
        grid=(GRID,),
        in_specs=[_blk(FW), _blk(FW), _blk(DEGW), _blk(DEGW), _blk(OUT)],
        out_specs=_blk(OUT),
        out_shape=jax.ShapeDtypeStruct((N, OUT), jnp.float32),
    )(p3A, p3B, dA, dB, r)
    return out
